# baseline (speedup 1.0000x reference)
import math

import jax
import jax.numpy as jnp
from jax.experimental import pallas as pl

R = 64
D = 256
M_REG = 8
N_NB = 6
G = 8
FB_ALPHA = 0.1
FB_SCALE = 1.0 / math.sqrt(M_REG)
NB_OFFS = (1, 63, 8, 56, 9, 55)
INV_SQRT_D = 1.0 / math.sqrt(D)


def _row_message(r, ht, coords_t, mask_t, w_blk, k_blk, q_blk, pt_blk,
                 wreg, bcos, bsin):
    f32 = jnp.float32

    onehot_r = (jax.lax.broadcasted_iota(jnp.int32, (1, R), 1) == r
                ).astype(f32)
    hr_col = jnp.sum(ht * onehot_r, axis=1, keepdims=True)
    coords_r = jnp.sum(coords_t * onehot_r, axis=1, keepdims=True)

    s_iota = jax.lax.broadcasted_iota(jnp.int32, (R, N_NB), 0)
    j_iota = jax.lax.broadcasted_iota(jnp.int32, (R, N_NB), 1)
    sel = jnp.zeros((R, N_NB), f32)
    for j, off in enumerate(NB_OFFS):
        idx = jax.lax.rem(r + off, R)
        sel = sel + ((s_iota == idx) & (j_iota == j)).astype(f32)

    hs_mat = jnp.dot(ht, sel, preferred_element_type=f32)
    m_sel = jnp.dot(w_blk, hs_mat, preferred_element_type=f32)
    k_sel = jnp.dot(k_blk, hs_mat, preferred_element_type=f32)

    col_iota = jax.lax.broadcasted_iota(jnp.int32, (1, N_NB), 1)
    msg_mat = jnp.zeros((D, N_NB), f32)
    k_mat = jnp.zeros((D, N_NB), f32)
    for j in range(N_NB):
        cmask = (col_iota == j).astype(f32)
        msg_mat = msg_mat + m_sel[j * D:(j + 1) * D, :] * cmask
        k_mat = k_mat + k_sel[j * D:(j + 1) * D, :] * cmask

    q_col = jnp.dot(q_blk, hr_col, preferred_element_type=f32)

    coords_s = jnp.dot(coords_t, sel, preferred_element_type=f32)
    delta = coords_r - coords_s
    phase = jnp.dot(wreg, delta, preferred_element_type=f32)
    b_row = FB_SCALE * (
        jnp.sum(jnp.cos(phase) * bcos, axis=0, keepdims=True)
        + jnp.sum(jnp.sin(phase) * bsin, axis=0, keepdims=True))
    msg_mat = msg_mat * (1.0 + FB_ALPHA * b_row)

    score_row = jnp.sum(q_col * k_mat, axis=0, keepdims=True) * INV_SQRT_D

    resid = msg_mat - hr_col
    p_mat = jax.nn.softplus(pt_blk)
    mah_row = jnp.sum(resid * resid * p_mat, axis=0, keepdims=True)
    rob_row = jnp.exp(-0.5 * mah_row)

    mask_row = jnp.dot(mask_t, sel, preferred_element_type=f32)

    neg_inf = f32(-jnp.inf)
    s_masked = jnp.where(mask_row > 0, score_row, neg_inf)
    any_m = jnp.max(mask_row, axis=1, keepdims=True)
    mx = jnp.max(s_masked, axis=1, keepdims=True)
    mx = jnp.where(any_m > 0, mx, 0.0)
    unn = jnp.exp(s_masked - mx)
    denom = jnp.where(any_m > 0,
                      jnp.sum(unn, axis=1, keepdims=True), 1.0)
    w_row = (unn / denom) * rob_row
    z = jnp.sum(w_row, axis=1, keepdims=True)
    w_row = jnp.where(z > 0, w_row / z, w_row)

    acc = jnp.sum(msg_mat * w_row, axis=1, keepdims=True)
    return acc, onehot_r


def _router_kernel(htc_ref, coords_t_ref, mask_t_ref, wa_ref, wb_ref,
                   ka_ref, kb_ref, q_ref, pt_ref, wreg_ref, bcos_ref,
                   bsin_ref, out_ref):
    i = pl.program_id(0)
    ht = htc_ref[...]
    coords_t = coords_t_ref[...]
    mask_t = mask_t_ref[...]
    wreg = wreg_ref[...]
    bcos = bcos_ref[...]
    bsin = bsin_ref[...]

    half = G // 2
    contrib = jnp.zeros((D, R), jnp.float32)
    union = jnp.zeros((1, R), jnp.float32)
    for g in range(G):
        r = i * G + g
        w_ref = wa_ref if g < half else wb_ref
        k_ref = ka_ref if g < half else kb_ref
        gg = g % half
        acc, onehot_r = _row_message(
            r, ht, coords_t, mask_t,
            w_ref[gg * N_NB * D:(gg + 1) * N_NB * D, :],
            k_ref[gg * N_NB * D:(gg + 1) * N_NB * D, :],
            q_ref[g], pt_ref[g], wreg, bcos, bsin)
        contrib = contrib + acc * onehot_r
        union = union + onehot_r

    out_ref[...] = jnp.where(union > 0, contrib, out_ref[...])


def kernel(H, reg_mask_prev, reg_coords, W_edge, K_edge, Q_lin, raw_P_edge,
           W_reg, beta_cos, beta_sin):
    HT = H.T
    coords_t = reg_coords.T
    mask_t = reg_mask_prev.astype(jnp.float32).reshape(1, R)
    PT = raw_P_edge.reshape(R, N_NB, D).transpose(0, 2, 1)
    W2 = W_edge.reshape(R * N_NB * D, D)
    K2 = K_edge.reshape(R * N_NB * D, D)
    bcos = beta_cos.reshape(M_REG, 1)
    bsin = beta_sin.reshape(M_REG, 1)

    out_t = pl.pallas_call(
        _router_kernel,
        grid=(R // G,),
        in_specs=[
            pl.BlockSpec((D, R), lambda i: (0, 0)),
            pl.BlockSpec((2, R), lambda i: (0, 0)),
            pl.BlockSpec((1, R), lambda i: (0, 0)),
            pl.BlockSpec((G * N_NB * D // 2, D),
                         lambda i: (2 * i, 0)),
            pl.BlockSpec((G * N_NB * D // 2, D),
                         lambda i: (2 * i + 1, 0)),
            pl.BlockSpec((G * N_NB * D // 2, D),
                         lambda i: (2 * i, 0)),
            pl.BlockSpec((G * N_NB * D // 2, D),
                         lambda i: (2 * i + 1, 0)),
            pl.BlockSpec((G, D, D), lambda i: (i, 0, 0)),
            pl.BlockSpec((G, D, N_NB), lambda i: (i, 0, 0)),
            pl.BlockSpec((M_REG, 2), lambda i: (0, 0)),
            pl.BlockSpec((M_REG, 1), lambda i: (0, 0)),
            pl.BlockSpec((M_REG, 1), lambda i: (0, 0)),
        ],
        out_specs=pl.BlockSpec((D, R), lambda i: (0, 0)),
        out_shape=jax.ShapeDtypeStruct((D, R), jnp.float32),
    )(HT, coords_t, mask_t, W2, W2, K2, K2, Q_lin, PT, W_reg, bcos, bsin)
    return out_t.T


# speedup vs baseline: 1.0028x; 1.0028x over previous
import math

import jax
import jax.numpy as jnp
from jax.experimental import pallas as pl

R = 64
D = 256
M_REG = 8
N_NB = 6
G = 4
FB_ALPHA = 0.1
FB_SCALE = 1.0 / math.sqrt(M_REG)
NB_OFFS = (1, 63, 8, 56, 9, 55)
INV_SQRT_D = 1.0 / math.sqrt(D)


def _row_message(r, ht, coords_t, mask_t, w_blk, k_blk, q_blk, pt_blk,
                 wreg, bcos, bsin):
    f32 = jnp.float32

    onehot_r = (jax.lax.broadcasted_iota(jnp.int32, (1, R), 1) == r
                ).astype(f32)
    hr_col = jnp.sum(ht * onehot_r, axis=1, keepdims=True)
    coords_r = jnp.sum(coords_t * onehot_r, axis=1, keepdims=True)

    s_iota = jax.lax.broadcasted_iota(jnp.int32, (R, N_NB), 0)
    j_iota = jax.lax.broadcasted_iota(jnp.int32, (R, N_NB), 1)
    sel = jnp.zeros((R, N_NB), f32)
    for j, off in enumerate(NB_OFFS):
        idx = jax.lax.rem(r + off, R)
        sel = sel + ((s_iota == idx) & (j_iota == j)).astype(f32)

    hs_mat = jnp.dot(ht, sel, preferred_element_type=f32)
    m_sel = jnp.dot(w_blk, hs_mat, preferred_element_type=f32)
    k_sel = jnp.dot(k_blk, hs_mat, preferred_element_type=f32)

    col_iota = jax.lax.broadcasted_iota(jnp.int32, (1, N_NB), 1)
    msg_mat = jnp.zeros((D, N_NB), f32)
    k_mat = jnp.zeros((D, N_NB), f32)
    for j in range(N_NB):
        cmask = (col_iota == j).astype(f32)
        msg_mat = msg_mat + m_sel[j * D:(j + 1) * D, :] * cmask
        k_mat = k_mat + k_sel[j * D:(j + 1) * D, :] * cmask

    q_col = jnp.dot(q_blk, hr_col, preferred_element_type=f32)

    coords_s = jnp.dot(coords_t, sel, preferred_element_type=f32)
    delta = coords_r - coords_s
    phase = jnp.dot(wreg, delta, preferred_element_type=f32)
    b_row = FB_SCALE * (
        jnp.sum(jnp.cos(phase) * bcos, axis=0, keepdims=True)
        + jnp.sum(jnp.sin(phase) * bsin, axis=0, keepdims=True))
    msg_mat = msg_mat * (1.0 + FB_ALPHA * b_row)

    score_row = jnp.sum(q_col * k_mat, axis=0, keepdims=True) * INV_SQRT_D

    resid = msg_mat - hr_col
    p_mat = jax.nn.softplus(pt_blk)
    mah_row = jnp.sum(resid * resid * p_mat, axis=0, keepdims=True)
    rob_row = jnp.exp(-0.5 * mah_row)

    mask_row = jnp.dot(mask_t, sel, preferred_element_type=f32)

    neg_inf = f32(-jnp.inf)
    s_masked = jnp.where(mask_row > 0, score_row, neg_inf)
    any_m = jnp.max(mask_row, axis=1, keepdims=True)
    mx = jnp.max(s_masked, axis=1, keepdims=True)
    mx = jnp.where(any_m > 0, mx, 0.0)
    unn = jnp.exp(s_masked - mx)
    denom = jnp.where(any_m > 0,
                      jnp.sum(unn, axis=1, keepdims=True), 1.0)
    w_row = (unn / denom) * rob_row
    z = jnp.sum(w_row, axis=1, keepdims=True)
    w_row = jnp.where(z > 0, w_row / z, w_row)

    acc = jnp.sum(msg_mat * w_row, axis=1, keepdims=True)
    return acc, onehot_r


def _router_kernel(htc_ref, coords_t_ref, mask_t_ref, w_ref, k_ref,
                   q_ref, pt_ref, wreg_ref, bcos_ref, bsin_ref, out_ref):
    i = pl.program_id(0)
    ht = htc_ref[...]
    coords_t = coords_t_ref[...]
    mask_t = mask_t_ref[...]
    wreg = wreg_ref[...]
    bcos = bcos_ref[...]
    bsin = bsin_ref[...]

    contrib = jnp.zeros((D, R), jnp.float32)
    union = jnp.zeros((1, R), jnp.float32)
    for g in range(G):
        r = i * G + g
        acc, onehot_r = _row_message(
            r, ht, coords_t, mask_t,
            w_ref[g * N_NB * D:(g + 1) * N_NB * D, :],
            k_ref[g * N_NB * D:(g + 1) * N_NB * D, :],
            q_ref[g], pt_ref[g], wreg, bcos, bsin)
        contrib = contrib + acc * onehot_r
        union = union + onehot_r

    out_ref[...] = jnp.where(union > 0, contrib, out_ref[...])


def kernel(H, reg_mask_prev, reg_coords, W_edge, K_edge, Q_lin, raw_P_edge,
           W_reg, beta_cos, beta_sin):
    HT = H.T
    coords_t = reg_coords.T
    mask_t = reg_mask_prev.astype(jnp.float32).reshape(1, R)
    PT = raw_P_edge.reshape(R, N_NB, D).transpose(0, 2, 1)
    W2 = W_edge.reshape(R * N_NB * D, D)
    K2 = K_edge.reshape(R * N_NB * D, D)
    bcos = beta_cos.reshape(M_REG, 1)
    bsin = beta_sin.reshape(M_REG, 1)

    out_t = pl.pallas_call(
        _router_kernel,
        grid=(R // G,),
        in_specs=[
            pl.BlockSpec((D, R), lambda i: (0, 0)),
            pl.BlockSpec((2, R), lambda i: (0, 0)),
            pl.BlockSpec((1, R), lambda i: (0, 0)),
            pl.BlockSpec((G * N_NB * D, D), lambda i: (i, 0)),
            pl.BlockSpec((G * N_NB * D, D), lambda i: (i, 0)),
            pl.BlockSpec((G, D, D), lambda i: (i, 0, 0)),
            pl.BlockSpec((G, D, N_NB), lambda i: (i, 0, 0)),
            pl.BlockSpec((M_REG, 2), lambda i: (0, 0)),
            pl.BlockSpec((M_REG, 1), lambda i: (0, 0)),
            pl.BlockSpec((M_REG, 1), lambda i: (0, 0)),
        ],
        out_specs=pl.BlockSpec((D, R), lambda i: (0, 0)),
        out_shape=jax.ShapeDtypeStruct((D, R), jnp.float32),
    )(HT, coords_t, mask_t, W2, K2, Q_lin, PT, W_reg, bcos, bsin)
    return out_t.T
